# baseline (speedup 1.0000x reference)
import dataclasses
import functools

import jax
import jax.numpy as jnp
from jax import lax
from jax.experimental import pallas as pl
from jax.experimental.pallas import tpu as pltpu
from jax.experimental.pallas import tpu_sc as plsc

_NC = 2
_NS = 16
_L = 16
_CHUNK = 128


def _make_sc_call(B, D, n_workers, bpw, nchunk):
    mesh = plsc.VectorSubcoreMesh(
        core_axis_name="c", subcore_axis_name="s",
        num_cores=_NC, num_subcores=_NS)

    cp = pltpu.CompilerParams()
    if "needs_layout_passes" in pltpu.CompilerParams.__dataclass_fields__:
        cp = dataclasses.replace(cp, needs_layout_passes=False)
    if "use_tc_tiling_on_sc" in pltpu.CompilerParams.__dataclass_fields__:
        cp = dataclasses.replace(cp, use_tc_tiling_on_sc=False)

    @functools.partial(
        pl.kernel,
        compiler_params=cp,
        out_type=jax.ShapeDtypeStruct((n_workers, bpw), jnp.float32),
        mesh=mesh,
        scratch_types=[
            pltpu.VMEM((bpw * 3,), jnp.int32),
            pltpu.VMEM((bpw,), jnp.int32),
            pltpu.VMEM((bpw,), jnp.int32),
            pltpu.VMEM((bpw, D), jnp.float32),
            pltpu.VMEM((bpw, D), jnp.float32),
            pltpu.VMEM((bpw,), jnp.float32),
            pltpu.SemaphoreType.DMA,
            pltpu.SemaphoreType.DMA,
        ],
    )
    def run(pair_hbm, u_hbm, v_hbm, o_hbm,
            pair_v, idx_u, idx_v, urows, vrows, out_v, sem_u, sem_v):
        w = lax.axis_index("s") * _NC + lax.axis_index("c")
        pltpu.sync_copy(pair_hbm.at[w], pair_v)

        lane = lax.iota(jnp.int32, _L)
        for g in range(bpw // _L):
            rows3 = (g * _L * 3) + lane * 3
            idx_u[pl.ds(g * _L, _L)] = plsc.load_gather(pair_v, [rows3])
            idx_v[pl.ds(g * _L, _L)] = plsc.load_gather(pair_v, [rows3 + 1])

        copies = []
        for c in range(nchunk):
            copies.append(pltpu.async_copy(
                u_hbm.at[idx_u.at[pl.ds(c * _CHUNK, _CHUNK)]],
                urows.at[pl.ds(c * _CHUNK, _CHUNK)], sem_u))
            copies.append(pltpu.async_copy(
                v_hbm.at[idx_v.at[pl.ds(c * _CHUNK, _CHUNK)]],
                vrows.at[pl.ds(c * _CHUNK, _CHUNK)], sem_v))
        for cp_ in copies:
            cp_.wait()

        @pl.loop(0, bpw, step=_L)
        def _(r0):
            rows = r0 + lane
            acc0 = jnp.zeros((_L,), jnp.float32)
            acc1 = jnp.zeros((_L,), jnp.float32)
            for d in range(0, D, 2):
                c0 = jnp.full((_L,), d, jnp.int32)
                c1 = jnp.full((_L,), d + 1, jnp.int32)
                acc0 += (plsc.load_gather(urows, [rows, c0])
                         * plsc.load_gather(vrows, [rows, c0]))
                acc1 += (plsc.load_gather(urows, [rows, c1])
                         * plsc.load_gather(vrows, [rows, c1]))
            de = plsc.load_gather(pair_v, [rows * 3 + 2]).astype(jnp.float32)
            t = de * (acc0 + acc1)
            out_v[pl.ds(r0, _L)] = -1.0 / (1.0 + jnp.exp(-t))

        pltpu.sync_copy(out_v, o_hbm.at[w])

    return run


def kernel(pair, U, V):
    B = pair.shape[0]
    D = U.shape[1]
    n_workers = _NC * _NS
    bpw = B // n_workers
    nchunk = bpw // _CHUNK

    pair_flat = pair.reshape(n_workers, bpw * 3)

    run = _make_sc_call(B, D, n_workers, bpw, nchunk)
    out = run(pair_flat, U, V)
    return out.reshape(B, 1)


# speedup vs baseline: 4.0445x; 4.0445x over previous
import dataclasses
import functools

import jax
import jax.numpy as jnp
from jax import lax
from jax.experimental import pallas as pl
from jax.experimental.pallas import tpu as pltpu
from jax.experimental.pallas import tpu_sc as plsc

_NC = 2
_NS = 16
_L = 16
_NBUF = 8


def _make_sc_call(B, D, n_workers, bpw):
    mesh = plsc.VectorSubcoreMesh(
        core_axis_name="c", subcore_axis_name="s",
        num_cores=_NC, num_subcores=_NS)

    cp = pltpu.CompilerParams()
    if "needs_layout_passes" in pltpu.CompilerParams.__dataclass_fields__:
        cp = dataclasses.replace(cp, needs_layout_passes=False)

    tile_bufs = [pltpu.VMEM((D, 128), jnp.float32)
                 for _ in range(2 * _NBUF)]

    @functools.partial(
        pl.kernel,
        compiler_params=cp,
        out_type=jax.ShapeDtypeStruct((n_workers, bpw), jnp.float32),
        mesh=mesh,
        scratch_types=[
            pltpu.VMEM((bpw * 2,), jnp.int32),
            pltpu.VMEM((bpw,), jnp.float32),
            pltpu.VMEM((D, bpw), jnp.float32),
            pltpu.VMEM((D, bpw), jnp.float32),
            pltpu.VMEM((bpw,), jnp.float32),
            *tile_bufs,
            pltpu.SemaphoreType.DMA,
            pltpu.SemaphoreType.DMA,
        ],
    )
    def run(ij_hbm, de_hbm, ut_hbm, vt_hbm, o_hbm,
            ij_v, de_v, utc, vtc, out_v, *rest):
        ubufs = rest[:_NBUF]
        vbufs = rest[_NBUF:2 * _NBUF]
        sem_u, sem_v = rest[2 * _NBUF:]

        w = lax.axis_index("s") * _NC + lax.axis_index("c")
        pltpu.sync_copy(ij_hbm.at[w], ij_v)
        pltpu.sync_copy(de_hbm.at[w], de_v)

        lane = lax.iota(jnp.int32, _L)
        lane2 = jnp.bitwise_and(lane, jnp.int32(1))

        def scalars(k):
            x = plsc.load_gather(ij_v, [2 * k + lane2])
            return x[0], x[1]

        def fire(k, slot):
            i, j = scalars(k)
            bi = pl.multiple_of(
                jnp.bitwise_and(i, jnp.int32(~127)), 128)
            bj = pl.multiple_of(
                jnp.bitwise_and(j, jnp.int32(~127)), 128)
            pltpu.async_copy(ut_hbm.at[:, pl.ds(bi, 128)], ubufs[slot], sem_u)
            pltpu.async_copy(vt_hbm.at[:, pl.ds(bj, 128)], vbufs[slot], sem_v)

        def wait(slot):
            pltpu.make_async_copy(
                ut_hbm.at[:, pl.ds(0, 128)], ubufs[slot], sem_u).wait()
            pltpu.make_async_copy(
                vt_hbm.at[:, pl.ds(0, 128)], vbufs[slot], sem_v).wait()

        def process(k, slot):
            i, j = scalars(k)
            ci = jnp.full((_L,), jnp.bitwise_and(i, jnp.int32(127)), jnp.int32)
            cj = jnp.full((_L,), jnp.bitwise_and(j, jnp.int32(127)), jnp.int32)
            kk = jnp.full((_L,), k, jnp.int32)
            u0 = plsc.load_gather(ubufs[slot], [lane, ci])
            u1 = plsc.load_gather(ubufs[slot], [lane + _L, ci])
            v0 = plsc.load_gather(vbufs[slot], [lane, cj])
            v1 = plsc.load_gather(vbufs[slot], [lane + _L, cj])
            plsc.store_scatter(utc, [lane, kk], u0)
            plsc.store_scatter(utc, [lane + _L, kk], u1)
            plsc.store_scatter(vtc, [lane, kk], v0)
            plsc.store_scatter(vtc, [lane + _L, kk], v1)

        for s in range(_NBUF):
            fire(s, s)

        @pl.loop(0, bpw, step=_NBUF)
        def _(k0):
            for s in range(_NBUF):
                k = k0 + s
                wait(s)
                process(k, s)
                fire(jnp.minimum(k + _NBUF, bpw - 1), s)

        for s in range(_NBUF):
            wait(s)

        @pl.loop(0, bpw, step=_L)
        def _(k0):
            acc0 = jnp.zeros((_L,), jnp.float32)
            acc1 = jnp.zeros((_L,), jnp.float32)
            for d in range(0, D, 2):
                acc0 += utc[d, pl.ds(k0, _L)] * vtc[d, pl.ds(k0, _L)]
                acc1 += utc[d + 1, pl.ds(k0, _L)] * vtc[d + 1, pl.ds(k0, _L)]
            t = de_v[pl.ds(k0, _L)] * (acc0 + acc1)
            out_v[pl.ds(k0, _L)] = -1.0 / (1.0 + jnp.exp(-t))

        pltpu.sync_copy(out_v, o_hbm.at[w])

    return run


def kernel(pair, U, V):
    B = pair.shape[0]
    D = U.shape[1]
    n_workers = _NC * _NS
    bpw = B // n_workers

    ij = pair[:, :2].reshape(n_workers, bpw * 2)
    de = pair[:, 2].astype(jnp.float32).reshape(n_workers, bpw)

    run = _make_sc_call(B, D, n_workers, bpw)
    out = run(ij, de, U.T, V.T)
    return out.reshape(B, 1)
